# baseline (speedup 1.0000x reference)
import math

import jax
import jax.numpy as jnp
from jax import lax
from jax.experimental import pallas as pl
from jax.experimental.pallas import tpu as pltpu
from jax.experimental.pallas import tpu_sc as plsc

_N = 10000
_H = 128
_E = 320000
_G = 128
_BN_EPS = 1e-5
_INV_BN = 1.0 / math.sqrt(1.0 + _BN_EPS)

_NC, _NS = 2, 16
_NW = _NC * _NS

_CH = 50
_CHUNKS = _E // _CH
_CPT = _CHUNKS // _NW
_NPAD = 10240

_EMB_CH = 80
_EMB_CHUNKS = _N // _EMB_CH
_EMB_PER_W = -(-_EMB_CHUNKS // _NW)

_mesh = plsc.VectorSubcoreMesh(
    core_axis_name="c", subcore_axis_name="s", num_cores=_NC, num_subcores=_NS
)


def _embed_body(z_hbm, tab_hbm, out_hbm, idx_v, rows_v, sem):
    wid = lax.axis_index("s") * _NC + lax.axis_index("c")
    for k in range(_EMB_PER_W):
        cid = wid * _EMB_PER_W + k

        @pl.when(cid < _EMB_CHUNKS)
        def _():
            off = pl.multiple_of(cid * _EMB_CH, _EMB_CH)
            pltpu.sync_copy(z_hbm.at[pl.ds(off, _EMB_CH)], idx_v)
            pltpu.async_copy(tab_hbm.at[idx_v], rows_v, sem).wait()
            pltpu.sync_copy(rows_v, out_hbm.at[pl.ds(off, _EMB_CH)])


_embed = pl.kernel(
    _embed_body,
    out_type=jax.ShapeDtypeStruct((_N, _H), jnp.float32),
    mesh=_mesh,
    scratch_types=[
        pltpu.VMEM((_EMB_CH,), jnp.int32),
        pltpu.VMEM((_EMB_CH, _H), jnp.float32),
        pltpu.SemaphoreType.DMA,
    ],
)


def _agg_body(x_hbm, ei_hbm, zero_hbm, out_hbm,
              ebuf0, ebuf1, rows0, rows1, agg_sh,
              esem0, esem1, sem0, sem1):
    c = lax.axis_index("c")
    s = lax.axis_index("s")

    @pl.when(s == 0)
    def _():
        pltpu.sync_copy(zero_hbm, agg_sh)

    plsc.subcore_barrier()

    chunk0 = pl.multiple_of((c * _NS + s) * _CPT, _CPT)

    def load_idx(g, ebuf, esem):
        return pltpu.async_copy(ei_hbm.at[chunk0 + g], ebuf, esem)

    def gather(ebuf, rows, sem):
        return pltpu.async_copy(x_hbm.at[ebuf.at[0]], rows, sem)

    def scatter(ebuf, rows):
        pltpu.sync_copy(rows, agg_sh.at[ebuf.at[1]], add=True)

    load_idx(0, ebuf0, esem0).wait()
    gather(ebuf0, rows0, sem0)
    load_idx(1, ebuf1, esem1)

    @pl.loop(0, _CPT, step=2)
    def _(g2):
        pltpu.make_async_copy(ei_hbm.at[chunk0 + g2 + 1], ebuf1, esem1).wait()
        gather(ebuf1, rows1, sem1)
        pltpu.make_async_copy(x_hbm.at[ebuf0.at[0]], rows0, sem0).wait()
        scatter(ebuf0, rows0)

        @pl.when(g2 + 2 < _CPT)
        def _():
            load_idx(g2 + 2, ebuf0, esem0)

        pltpu.make_async_copy(x_hbm.at[ebuf1.at[0]], rows1, sem1).wait()
        scatter(ebuf1, rows1)

        @pl.when(g2 + 3 < _CPT)
        def _():
            load_idx(g2 + 3, ebuf1, esem1)

        @pl.when(g2 + 2 < _CPT)
        def _():
            pltpu.make_async_copy(ei_hbm.at[chunk0 + g2 + 2], ebuf0, esem0).wait()
            gather(ebuf0, rows0, sem0)

    plsc.subcore_barrier()
    rpt = _NPAD // _NS
    pltpu.sync_copy(
        agg_sh.at[pl.ds(s * rpt, rpt)],
        out_hbm.at[c, pl.ds(s * rpt, rpt)],
    )


_agg = pl.kernel(
    _agg_body,
    out_type=jax.ShapeDtypeStruct((_NC, _NPAD, _H), jnp.float32),
    mesh=_mesh,
    scratch_types=[
        pltpu.VMEM((2, _CH), jnp.int32),
        pltpu.VMEM((2, _CH), jnp.int32),
        pltpu.VMEM((_CH, _H), jnp.float32),
        pltpu.VMEM((_CH, _H), jnp.float32),
        pltpu.VMEM_SHARED((_NPAD, _H), jnp.float32),
        pltpu.SemaphoreType.DMA,
        pltpu.SemaphoreType.DMA,
        pltpu.SemaphoreType.DMA,
        pltpu.SemaphoreType.DMA,
    ],
)


_BM = 1000


def _mlp_body(x_ref, agg_ref, w1_ref, b1_ref, w2_ref, b2_ref, g_ref, be_ref, out_ref):
    h = x_ref[...] + agg_ref[0] + agg_ref[1]
    h = jnp.maximum(jnp.dot(h, w1_ref[...], preferred_element_type=jnp.float32) + b1_ref[...], 0.0)
    h = jnp.maximum(jnp.dot(h, w2_ref[...], preferred_element_type=jnp.float32) + b2_ref[...], 0.0)
    out_ref[...] = h * (g_ref[...] * _INV_BN) + be_ref[...]


_mlp = pl.pallas_call(
    _mlp_body,
    grid=(_N // _BM,),
    in_specs=[
        pl.BlockSpec((_BM, _H), lambda i: (i, 0)),
        pl.BlockSpec((_NC, _BM, _H), lambda i: (0, i, 0)),
        pl.BlockSpec((_H, _H), lambda i: (0, 0)),
        pl.BlockSpec((1, _H), lambda i: (0, 0)),
        pl.BlockSpec((_H, _H), lambda i: (0, 0)),
        pl.BlockSpec((1, _H), lambda i: (0, 0)),
        pl.BlockSpec((1, _H), lambda i: (0, 0)),
        pl.BlockSpec((1, _H), lambda i: (0, 0)),
    ],
    out_specs=pl.BlockSpec((_BM, _H), lambda i: (i, 0)),
    out_shape=jax.ShapeDtypeStruct((_N, _H), jnp.float32),
)


def _pool_body(adj_ref, batch_ref, x1_ref, x2_ref, x3_ref, wm1_ref, bm1_ref,
               wm2_ref, bm2_ref, out_ref, pooled_s, cnt_s):
    i = pl.program_id(0)

    @pl.when(i == 0)
    def _():
        pooled_s[...] = jnp.zeros_like(pooled_s)
        cnt_s[...] = jnp.zeros_like(cnt_s)

    onehot = (batch_ref[...] == lax.broadcasted_iota(jnp.int32, (_BM, _G), 1)).astype(jnp.float32)
    h = jnp.concatenate([x1_ref[...], x2_ref[...], x3_ref[...]], axis=1)
    pooled_s[...] += lax.dot_general(
        onehot, h, (((0,), (0,)), ((), ())), preferred_element_type=jnp.float32)
    cnt_s[...] += lax.dot_general(
        onehot, jnp.ones((_BM, 1), jnp.float32), (((0,), (0,)), ((), ())),
        preferred_element_type=jnp.float32)

    @pl.when(i == _N // _BM - 1)
    def _():
        cnt = cnt_s[...] + adj_ref[0, 0]
        mean = pooled_s[...] / jnp.maximum(cnt, 1.0)
        hm = jnp.maximum(
            jnp.dot(mean, wm1_ref[...], preferred_element_type=jnp.float32) + bm1_ref[...], 0.0)
        out_ref[...] = jnp.dot(hm, wm2_ref[...], preferred_element_type=jnp.float32) + bm2_ref[...]


_pool = pl.pallas_call(
    _pool_body,
    grid=(_N // _BM,),
    in_specs=[
        pl.BlockSpec((1, 1), lambda i: (0, 0)),
        pl.BlockSpec((_BM, 1), lambda i: (i, 0)),
        pl.BlockSpec((_BM, _H), lambda i: (i, 0)),
        pl.BlockSpec((_BM, _H), lambda i: (i, 0)),
        pl.BlockSpec((_BM, _H), lambda i: (i, 0)),
        pl.BlockSpec((3 * _H, _H), lambda i: (0, 0)),
        pl.BlockSpec((1, _H), lambda i: (0, 0)),
        pl.BlockSpec((_H, 1), lambda i: (0, 0)),
        pl.BlockSpec((1, 1), lambda i: (0, 0)),
    ],
    out_specs=pl.BlockSpec((_G, 1), lambda i: (0, 0)),
    out_shape=jax.ShapeDtypeStruct((_G, 1), jnp.float32),
    scratch_shapes=[
        pltpu.VMEM((_G, 3 * _H), jnp.float32),
        pltpu.VMEM((_G, 1), jnp.float32),
    ],
)


def kernel(num_nodes, z, edge_index, batch, z_table,
           W1_0, b1_0, W2_0, b2_0, g_0, be_0,
           W1_1, b1_1, W2_1, b2_1, g_1, be_1,
           W1_2, b1_2, W2_2, b2_2, g_2, be_2,
           Wm1, bm1, Wm2, bm2):
    ei = edge_index.astype(jnp.int32).reshape(2, _CHUNKS, _CH).transpose(1, 0, 2)
    zeros = jnp.zeros((_NPAD, _H), jnp.float32)

    x = _embed(z.astype(jnp.int32), z_table)

    layer_params = [
        (W1_0, b1_0, W2_0, b2_0, g_0, be_0),
        (W1_1, b1_1, W2_1, b2_1, g_1, be_1),
        (W1_2, b1_2, W2_2, b2_2, g_2, be_2),
    ]
    xs = []
    for (w1, b1, w2, b2, g, be) in layer_params:
        agg = _agg(x, ei, zeros)
        x = _mlp(x, agg, w1, b1.reshape(1, _H), w2, b2.reshape(1, _H),
                 g.reshape(1, _H), be.reshape(1, _H))
        xs.append(x)

    adj = (jnp.asarray(num_nodes, jnp.int32) - _N).astype(jnp.float32).reshape(1, 1)
    out = _pool(adj, batch.astype(jnp.int32).reshape(_N, 1), xs[0], xs[1], xs[2],
                Wm1, bm1.reshape(1, _H), Wm2, bm2.reshape(1, 1))
    return out


# speedup vs baseline: 1.3505x; 1.3505x over previous
import math

import jax
import jax.numpy as jnp
from jax import lax
from jax.experimental import pallas as pl
from jax.experimental.pallas import tpu as pltpu
from jax.experimental.pallas import tpu_sc as plsc

_N = 10000
_H = 128
_E = 320000
_G = 128
_BN_EPS = 1e-5
_INV_BN = 1.0 / math.sqrt(1.0 + _BN_EPS)

_NC, _NS = 2, 16
_NW = _NC * _NS

_CH = 100
_CHUNKS = _E // _CH
_CPT = _CHUNKS // _NW
_NPAD = 10240

_EMB_CH = 80
_EMB_CHUNKS = _N // _EMB_CH
_EMB_PER_W = -(-_EMB_CHUNKS // _NW)

_mesh = plsc.VectorSubcoreMesh(
    core_axis_name="c", subcore_axis_name="s", num_cores=_NC, num_subcores=_NS
)


def _embed_body(z_hbm, tab_hbm, out_hbm, idx_v, rows_v, sem):
    wid = lax.axis_index("s") * _NC + lax.axis_index("c")
    for k in range(_EMB_PER_W):
        cid = wid * _EMB_PER_W + k

        @pl.when(cid < _EMB_CHUNKS)
        def _():
            off = pl.multiple_of(cid * _EMB_CH, _EMB_CH)
            pltpu.sync_copy(z_hbm.at[pl.ds(off, _EMB_CH)], idx_v)
            pltpu.async_copy(tab_hbm.at[idx_v], rows_v, sem).wait()
            pltpu.sync_copy(rows_v, out_hbm.at[pl.ds(off, _EMB_CH)])


_embed = pl.kernel(
    _embed_body,
    out_type=jax.ShapeDtypeStruct((_N, _H), jnp.float32),
    mesh=_mesh,
    scratch_types=[
        pltpu.VMEM((_EMB_CH,), jnp.int32),
        pltpu.VMEM((_EMB_CH, _H), jnp.float32),
        pltpu.SemaphoreType.DMA,
    ],
)


def _agg_body(x_hbm, ei_hbm, zero_hbm, out_hbm,
              ebuf0, ebuf1, rows0, rows1, agg_sh,
              esem0, esem1, sem0, sem1):
    c = lax.axis_index("c")
    s = lax.axis_index("s")

    @pl.when(s == 0)
    def _():
        pltpu.sync_copy(zero_hbm, agg_sh)

    plsc.subcore_barrier()

    chunk0 = pl.multiple_of((c * _NS + s) * _CPT, _CPT)

    def load_idx(g, ebuf, esem):
        return pltpu.async_copy(ei_hbm.at[chunk0 + g], ebuf, esem)

    def gather(ebuf, rows, sem):
        return pltpu.async_copy(x_hbm.at[ebuf.at[0]], rows, sem)

    def scatter(ebuf, rows):
        pltpu.sync_copy(rows, agg_sh.at[ebuf.at[1]], add=True)

    load_idx(0, ebuf0, esem0).wait()
    gather(ebuf0, rows0, sem0)
    load_idx(1, ebuf1, esem1)

    @pl.loop(0, _CPT, step=2)
    def _(g2):
        pltpu.make_async_copy(ei_hbm.at[chunk0 + g2 + 1], ebuf1, esem1).wait()
        gather(ebuf1, rows1, sem1)
        pltpu.make_async_copy(x_hbm.at[ebuf0.at[0]], rows0, sem0).wait()
        scatter(ebuf0, rows0)

        @pl.when(g2 + 2 < _CPT)
        def _():
            load_idx(g2 + 2, ebuf0, esem0)

        pltpu.make_async_copy(x_hbm.at[ebuf1.at[0]], rows1, sem1).wait()
        scatter(ebuf1, rows1)

        @pl.when(g2 + 3 < _CPT)
        def _():
            load_idx(g2 + 3, ebuf1, esem1)

        @pl.when(g2 + 2 < _CPT)
        def _():
            pltpu.make_async_copy(ei_hbm.at[chunk0 + g2 + 2], ebuf0, esem0).wait()
            gather(ebuf0, rows0, sem0)

    plsc.subcore_barrier()
    rpt = _NPAD // _NS
    pltpu.sync_copy(
        agg_sh.at[pl.ds(s * rpt, rpt)],
        out_hbm.at[c, pl.ds(s * rpt, rpt)],
    )


_agg = pl.kernel(
    _agg_body,
    out_type=jax.ShapeDtypeStruct((_NC, _NPAD, _H), jnp.float32),
    mesh=_mesh,
    scratch_types=[
        pltpu.VMEM((2, _CH), jnp.int32),
        pltpu.VMEM((2, _CH), jnp.int32),
        pltpu.VMEM((_CH, _H), jnp.float32),
        pltpu.VMEM((_CH, _H), jnp.float32),
        pltpu.VMEM_SHARED((_NPAD, _H), jnp.float32),
        pltpu.SemaphoreType.DMA,
        pltpu.SemaphoreType.DMA,
        pltpu.SemaphoreType.DMA,
        pltpu.SemaphoreType.DMA,
    ],
)


_BM = 1000


def _mlp_body(x_ref, agg_ref, w1_ref, b1_ref, w2_ref, b2_ref, g_ref, be_ref, out_ref):
    h = x_ref[...] + agg_ref[0] + agg_ref[1]
    h = jnp.maximum(jnp.dot(h, w1_ref[...], preferred_element_type=jnp.float32) + b1_ref[...], 0.0)
    h = jnp.maximum(jnp.dot(h, w2_ref[...], preferred_element_type=jnp.float32) + b2_ref[...], 0.0)
    out_ref[...] = h * (g_ref[...] * _INV_BN) + be_ref[...]


_mlp = pl.pallas_call(
    _mlp_body,
    grid=(_N // _BM,),
    in_specs=[
        pl.BlockSpec((_BM, _H), lambda i: (i, 0)),
        pl.BlockSpec((_NC, _BM, _H), lambda i: (0, i, 0)),
        pl.BlockSpec((_H, _H), lambda i: (0, 0)),
        pl.BlockSpec((1, _H), lambda i: (0, 0)),
        pl.BlockSpec((_H, _H), lambda i: (0, 0)),
        pl.BlockSpec((1, _H), lambda i: (0, 0)),
        pl.BlockSpec((1, _H), lambda i: (0, 0)),
        pl.BlockSpec((1, _H), lambda i: (0, 0)),
    ],
    out_specs=pl.BlockSpec((_BM, _H), lambda i: (i, 0)),
    out_shape=jax.ShapeDtypeStruct((_N, _H), jnp.float32),
)


def _pool_body(adj_ref, batch_ref, x1_ref, x2_ref, x3_ref, wm1_ref, bm1_ref,
               wm2_ref, bm2_ref, out_ref, pooled_s, cnt_s):
    i = pl.program_id(0)

    @pl.when(i == 0)
    def _():
        pooled_s[...] = jnp.zeros_like(pooled_s)
        cnt_s[...] = jnp.zeros_like(cnt_s)

    onehot = (batch_ref[...] == lax.broadcasted_iota(jnp.int32, (_BM, _G), 1)).astype(jnp.float32)
    h = jnp.concatenate([x1_ref[...], x2_ref[...], x3_ref[...]], axis=1)
    pooled_s[...] += lax.dot_general(
        onehot, h, (((0,), (0,)), ((), ())), preferred_element_type=jnp.float32)
    cnt_s[...] += lax.dot_general(
        onehot, jnp.ones((_BM, 1), jnp.float32), (((0,), (0,)), ((), ())),
        preferred_element_type=jnp.float32)

    @pl.when(i == _N // _BM - 1)
    def _():
        cnt = cnt_s[...] + adj_ref[0, 0]
        mean = pooled_s[...] / jnp.maximum(cnt, 1.0)
        hm = jnp.maximum(
            jnp.dot(mean, wm1_ref[...], preferred_element_type=jnp.float32) + bm1_ref[...], 0.0)
        out_ref[...] = jnp.dot(hm, wm2_ref[...], preferred_element_type=jnp.float32) + bm2_ref[...]


_pool = pl.pallas_call(
    _pool_body,
    grid=(_N // _BM,),
    in_specs=[
        pl.BlockSpec((1, 1), lambda i: (0, 0)),
        pl.BlockSpec((_BM, 1), lambda i: (i, 0)),
        pl.BlockSpec((_BM, _H), lambda i: (i, 0)),
        pl.BlockSpec((_BM, _H), lambda i: (i, 0)),
        pl.BlockSpec((_BM, _H), lambda i: (i, 0)),
        pl.BlockSpec((3 * _H, _H), lambda i: (0, 0)),
        pl.BlockSpec((1, _H), lambda i: (0, 0)),
        pl.BlockSpec((_H, 1), lambda i: (0, 0)),
        pl.BlockSpec((1, 1), lambda i: (0, 0)),
    ],
    out_specs=pl.BlockSpec((_G, 1), lambda i: (0, 0)),
    out_shape=jax.ShapeDtypeStruct((_G, 1), jnp.float32),
    scratch_shapes=[
        pltpu.VMEM((_G, 3 * _H), jnp.float32),
        pltpu.VMEM((_G, 1), jnp.float32),
    ],
)


def kernel(num_nodes, z, edge_index, batch, z_table,
           W1_0, b1_0, W2_0, b2_0, g_0, be_0,
           W1_1, b1_1, W2_1, b2_1, g_1, be_1,
           W1_2, b1_2, W2_2, b2_2, g_2, be_2,
           Wm1, bm1, Wm2, bm2):
    ei = edge_index.astype(jnp.int32).reshape(2, _CHUNKS, _CH).transpose(1, 0, 2)
    zeros = jnp.zeros((_NPAD, _H), jnp.float32)

    x = _embed(z.astype(jnp.int32), z_table)

    layer_params = [
        (W1_0, b1_0, W2_0, b2_0, g_0, be_0),
        (W1_1, b1_1, W2_1, b2_1, g_1, be_1),
        (W1_2, b1_2, W2_2, b2_2, g_2, be_2),
    ]
    xs = []
    for (w1, b1, w2, b2, g, be) in layer_params:
        agg = _agg(x, ei, zeros)
        x = _mlp(x, agg, w1, b1.reshape(1, _H), w2, b2.reshape(1, _H),
                 g.reshape(1, _H), be.reshape(1, _H))
        xs.append(x)

    adj = (jnp.asarray(num_nodes, jnp.int32) - _N).astype(jnp.float32).reshape(1, 1)
    out = _pool(adj, batch.astype(jnp.int32).reshape(_N, 1), xs[0], xs[1], xs[2],
                Wm1, bm1.reshape(1, _H), Wm2, bm2.reshape(1, 1))
    return out


# speedup vs baseline: 1.3764x; 1.0192x over previous
import math

import jax
import jax.numpy as jnp
from jax import lax
from jax.experimental import pallas as pl
from jax.experimental.pallas import tpu as pltpu
from jax.experimental.pallas import tpu_sc as plsc

_N = 10000
_H = 128
_E = 320000
_G = 128
_BN_EPS = 1e-5
_INV_BN = 1.0 / math.sqrt(1.0 + _BN_EPS)

_NC, _NS = 2, 16
_NW = _NC * _NS

_CH = 125
_CHUNKS = _E // _CH
_CPT = _CHUNKS // _NW
_NPAD = 10240

_EMB_CH = 80
_EMB_CHUNKS = _N // _EMB_CH
_EMB_PER_W = -(-_EMB_CHUNKS // _NW)

_mesh = plsc.VectorSubcoreMesh(
    core_axis_name="c", subcore_axis_name="s", num_cores=_NC, num_subcores=_NS
)


def _embed_body(z_hbm, tab_hbm, out_hbm, idx_v, rows_v, sem):
    wid = lax.axis_index("s") * _NC + lax.axis_index("c")
    for k in range(_EMB_PER_W):
        cid = wid * _EMB_PER_W + k

        @pl.when(cid < _EMB_CHUNKS)
        def _():
            off = pl.multiple_of(cid * _EMB_CH, _EMB_CH)
            pltpu.sync_copy(z_hbm.at[pl.ds(off, _EMB_CH)], idx_v)
            pltpu.async_copy(tab_hbm.at[idx_v], rows_v, sem).wait()
            pltpu.sync_copy(rows_v, out_hbm.at[pl.ds(off, _EMB_CH)])


_embed = pl.kernel(
    _embed_body,
    out_type=jax.ShapeDtypeStruct((_N, _H), jnp.float32),
    mesh=_mesh,
    scratch_types=[
        pltpu.VMEM((_EMB_CH,), jnp.int32),
        pltpu.VMEM((_EMB_CH, _H), jnp.float32),
        pltpu.SemaphoreType.DMA,
    ],
)


def _agg_body(x_hbm, ei_hbm, zero_hbm, out_hbm,
              ebuf0, ebuf1, rows0, rows1, agg_sh,
              esem0, esem1, sem0, sem1):
    c = lax.axis_index("c")
    s = lax.axis_index("s")

    @pl.when(s == 0)
    def _():
        pltpu.sync_copy(zero_hbm, agg_sh)

    plsc.subcore_barrier()

    chunk0 = pl.multiple_of((c * _NS + s) * _CPT, _CPT)

    def load_idx(g, ebuf, esem):
        return pltpu.async_copy(ei_hbm.at[chunk0 + g], ebuf, esem)

    def gather(ebuf, rows, sem):
        return pltpu.async_copy(x_hbm.at[ebuf.at[0]], rows, sem)

    def scatter(ebuf, rows):
        pltpu.sync_copy(rows, agg_sh.at[ebuf.at[1]], add=True)

    load_idx(0, ebuf0, esem0).wait()
    gather(ebuf0, rows0, sem0)
    load_idx(1, ebuf1, esem1)

    @pl.loop(0, _CPT, step=2)
    def _(g2):
        pltpu.make_async_copy(ei_hbm.at[chunk0 + g2 + 1], ebuf1, esem1).wait()
        gather(ebuf1, rows1, sem1)
        pltpu.make_async_copy(x_hbm.at[ebuf0.at[0]], rows0, sem0).wait()
        scatter(ebuf0, rows0)

        @pl.when(g2 + 2 < _CPT)
        def _():
            load_idx(g2 + 2, ebuf0, esem0)

        pltpu.make_async_copy(x_hbm.at[ebuf1.at[0]], rows1, sem1).wait()
        scatter(ebuf1, rows1)

        @pl.when(g2 + 3 < _CPT)
        def _():
            load_idx(g2 + 3, ebuf1, esem1)

        @pl.when(g2 + 2 < _CPT)
        def _():
            pltpu.make_async_copy(ei_hbm.at[chunk0 + g2 + 2], ebuf0, esem0).wait()
            gather(ebuf0, rows0, sem0)

    plsc.subcore_barrier()
    rpt = _NPAD // _NS
    pltpu.sync_copy(
        agg_sh.at[pl.ds(s * rpt, rpt)],
        out_hbm.at[c, pl.ds(s * rpt, rpt)],
    )


_agg = pl.kernel(
    _agg_body,
    out_type=jax.ShapeDtypeStruct((_NC, _NPAD, _H), jnp.float32),
    mesh=_mesh,
    scratch_types=[
        pltpu.VMEM((2, _CH), jnp.int32),
        pltpu.VMEM((2, _CH), jnp.int32),
        pltpu.VMEM((_CH, _H), jnp.float32),
        pltpu.VMEM((_CH, _H), jnp.float32),
        pltpu.VMEM_SHARED((_NPAD, _H), jnp.float32),
        pltpu.SemaphoreType.DMA,
        pltpu.SemaphoreType.DMA,
        pltpu.SemaphoreType.DMA,
        pltpu.SemaphoreType.DMA,
    ],
)


_BM = 1000


def _mlp_body(x_ref, agg_ref, w1_ref, b1_ref, w2_ref, b2_ref, g_ref, be_ref, out_ref):
    h = x_ref[...] + agg_ref[0] + agg_ref[1]
    h = jnp.maximum(jnp.dot(h, w1_ref[...], preferred_element_type=jnp.float32) + b1_ref[...], 0.0)
    h = jnp.maximum(jnp.dot(h, w2_ref[...], preferred_element_type=jnp.float32) + b2_ref[...], 0.0)
    out_ref[...] = h * (g_ref[...] * _INV_BN) + be_ref[...]


_mlp = pl.pallas_call(
    _mlp_body,
    grid=(_N // _BM,),
    in_specs=[
        pl.BlockSpec((_BM, _H), lambda i: (i, 0)),
        pl.BlockSpec((_NC, _BM, _H), lambda i: (0, i, 0)),
        pl.BlockSpec((_H, _H), lambda i: (0, 0)),
        pl.BlockSpec((1, _H), lambda i: (0, 0)),
        pl.BlockSpec((_H, _H), lambda i: (0, 0)),
        pl.BlockSpec((1, _H), lambda i: (0, 0)),
        pl.BlockSpec((1, _H), lambda i: (0, 0)),
        pl.BlockSpec((1, _H), lambda i: (0, 0)),
    ],
    out_specs=pl.BlockSpec((_BM, _H), lambda i: (i, 0)),
    out_shape=jax.ShapeDtypeStruct((_N, _H), jnp.float32),
)


def _pool_body(adj_ref, batch_ref, x1_ref, x2_ref, x3_ref, wm1_ref, bm1_ref,
               wm2_ref, bm2_ref, out_ref, pooled_s, cnt_s):
    i = pl.program_id(0)

    @pl.when(i == 0)
    def _():
        pooled_s[...] = jnp.zeros_like(pooled_s)
        cnt_s[...] = jnp.zeros_like(cnt_s)

    onehot = (batch_ref[...] == lax.broadcasted_iota(jnp.int32, (_BM, _G), 1)).astype(jnp.float32)
    h = jnp.concatenate([x1_ref[...], x2_ref[...], x3_ref[...]], axis=1)
    pooled_s[...] += lax.dot_general(
        onehot, h, (((0,), (0,)), ((), ())), preferred_element_type=jnp.float32)
    cnt_s[...] += lax.dot_general(
        onehot, jnp.ones((_BM, 1), jnp.float32), (((0,), (0,)), ((), ())),
        preferred_element_type=jnp.float32)

    @pl.when(i == _N // _BM - 1)
    def _():
        cnt = cnt_s[...] + adj_ref[0, 0]
        mean = pooled_s[...] / jnp.maximum(cnt, 1.0)
        hm = jnp.maximum(
            jnp.dot(mean, wm1_ref[...], preferred_element_type=jnp.float32) + bm1_ref[...], 0.0)
        out_ref[...] = jnp.dot(hm, wm2_ref[...], preferred_element_type=jnp.float32) + bm2_ref[...]


_pool = pl.pallas_call(
    _pool_body,
    grid=(_N // _BM,),
    in_specs=[
        pl.BlockSpec((1, 1), lambda i: (0, 0)),
        pl.BlockSpec((_BM, 1), lambda i: (i, 0)),
        pl.BlockSpec((_BM, _H), lambda i: (i, 0)),
        pl.BlockSpec((_BM, _H), lambda i: (i, 0)),
        pl.BlockSpec((_BM, _H), lambda i: (i, 0)),
        pl.BlockSpec((3 * _H, _H), lambda i: (0, 0)),
        pl.BlockSpec((1, _H), lambda i: (0, 0)),
        pl.BlockSpec((_H, 1), lambda i: (0, 0)),
        pl.BlockSpec((1, 1), lambda i: (0, 0)),
    ],
    out_specs=pl.BlockSpec((_G, 1), lambda i: (0, 0)),
    out_shape=jax.ShapeDtypeStruct((_G, 1), jnp.float32),
    scratch_shapes=[
        pltpu.VMEM((_G, 3 * _H), jnp.float32),
        pltpu.VMEM((_G, 1), jnp.float32),
    ],
)


def kernel(num_nodes, z, edge_index, batch, z_table,
           W1_0, b1_0, W2_0, b2_0, g_0, be_0,
           W1_1, b1_1, W2_1, b2_1, g_1, be_1,
           W1_2, b1_2, W2_2, b2_2, g_2, be_2,
           Wm1, bm1, Wm2, bm2):
    ei = edge_index.astype(jnp.int32).reshape(2, _CHUNKS, _CH).transpose(1, 0, 2)
    zeros = jnp.zeros((_NPAD, _H), jnp.float32)

    x = _embed(z.astype(jnp.int32), z_table)

    layer_params = [
        (W1_0, b1_0, W2_0, b2_0, g_0, be_0),
        (W1_1, b1_1, W2_1, b2_1, g_1, be_1),
        (W1_2, b1_2, W2_2, b2_2, g_2, be_2),
    ]
    xs = []
    for (w1, b1, w2, b2, g, be) in layer_params:
        agg = _agg(x, ei, zeros)
        x = _mlp(x, agg, w1, b1.reshape(1, _H), w2, b2.reshape(1, _H),
                 g.reshape(1, _H), be.reshape(1, _H))
        xs.append(x)

    adj = (jnp.asarray(num_nodes, jnp.int32) - _N).astype(jnp.float32).reshape(1, 1)
    out = _pool(adj, batch.astype(jnp.int32).reshape(_N, 1), xs[0], xs[1], xs[2],
                Wm1, bm1.reshape(1, _H), Wm2, bm2.reshape(1, 1))
    return out


# speedup vs baseline: 1.3770x; 1.0005x over previous
import math

import jax
import jax.numpy as jnp
from jax import lax
from jax.experimental import pallas as pl
from jax.experimental.pallas import tpu as pltpu
from jax.experimental.pallas import tpu_sc as plsc

_N = 10000
_H = 128
_E = 320000
_G = 128
_BN_EPS = 1e-5
_INV_BN = 1.0 / math.sqrt(1.0 + _BN_EPS)

_NC, _NS = 2, 16
_NW = _NC * _NS

_CH = 50
_CHUNKS = _E // _CH
_CPT = _CHUNKS // _NW
_NPAD = 10240

_EMB_CH = 80
_EMB_CHUNKS = _N // _EMB_CH
_EMB_PER_W = -(-_EMB_CHUNKS // _NW)

_mesh = plsc.VectorSubcoreMesh(
    core_axis_name="c", subcore_axis_name="s", num_cores=_NC, num_subcores=_NS
)


def _embed_body(z_hbm, tab_hbm, out_hbm, idx_v, rows_v, sem):
    wid = lax.axis_index("s") * _NC + lax.axis_index("c")
    for k in range(_EMB_PER_W):
        cid = wid * _EMB_PER_W + k

        @pl.when(cid < _EMB_CHUNKS)
        def _():
            off = pl.multiple_of(cid * _EMB_CH, _EMB_CH)
            pltpu.sync_copy(z_hbm.at[pl.ds(off, _EMB_CH)], idx_v)
            pltpu.async_copy(tab_hbm.at[idx_v], rows_v, sem).wait()
            pltpu.sync_copy(rows_v, out_hbm.at[pl.ds(off, _EMB_CH)])


_embed = pl.kernel(
    _embed_body,
    out_type=jax.ShapeDtypeStruct((_N, _H), jnp.float32),
    mesh=_mesh,
    scratch_types=[
        pltpu.VMEM((_EMB_CH,), jnp.int32),
        pltpu.VMEM((_EMB_CH, _H), jnp.float32),
        pltpu.SemaphoreType.DMA,
    ],
)


def _agg_body(x_hbm, ei_hbm, zero_hbm, out_hbm, *refs):
    ebuf = refs[0:4]
    rows = refs[4:8]
    agg_sh = refs[8]
    esem = refs[9:13]
    gsem = refs[13:17]
    ssem = refs[17:21]

    c = lax.axis_index("c")
    s = lax.axis_index("s")

    @pl.when(s == 0)
    def _():
        pltpu.sync_copy(zero_hbm, agg_sh)

    plsc.subcore_barrier()

    chunk0 = (c * _NS + s) * _CPT

    def load_idx(g, j):
        return pltpu.async_copy(ei_hbm.at[chunk0 + g], ebuf[j], esem[j])

    def gather(g, j):
        return pltpu.async_copy(x_hbm.at[ebuf[j].at[0]], rows[j], gsem[j])

    def wait_gather(j):
        pltpu.make_async_copy(x_hbm.at[ebuf[j].at[0]], rows[j], gsem[j]).wait()

    def scatter(j):
        return pltpu.async_copy(rows[j], agg_sh.at[ebuf[j].at[1]], ssem[j], add=True)

    def wait_scatter(j):
        pltpu.make_async_copy(rows[j], agg_sh.at[ebuf[j].at[1]], ssem[j]).wait()

    def wait_idx(j):
        pltpu.make_async_copy(ei_hbm.at[chunk0], ebuf[j], esem[j]).wait()

    load_idx(0, 0)
    load_idx(1, 1)
    wait_idx(0)
    gather(0, 0)
    wait_idx(1)
    gather(1, 1)

    @pl.loop(0, _CPT, step=4)
    def _(g4):
        for j in range(4):
            g = g4 + j
            j2 = (j + 2) % 4

            if j < 2:
                @pl.when(g >= 2)
                def _():
                    wait_scatter(j2)

                @pl.when(g + 2 < _CPT)
                def _():
                    load_idx(g + 2, j2)
            else:
                wait_scatter(j2)

                @pl.when(g + 2 < _CPT)
                def _():
                    load_idx(g + 2, j2)

            wait_gather(j)
            scatter(j)

            @pl.when(g + 2 < _CPT)
            def _():
                wait_idx(j2)
                gather(g + 2, j2)

    wait_scatter((_CPT - 2) % 4)
    wait_scatter((_CPT - 1) % 4)

    plsc.subcore_barrier()
    rpt = _NPAD // _NS
    pltpu.sync_copy(
        agg_sh.at[pl.ds(s * rpt, rpt)],
        out_hbm.at[c, pl.ds(s * rpt, rpt)],
    )


_agg = pl.kernel(
    _agg_body,
    out_type=jax.ShapeDtypeStruct((_NC, _NPAD, _H), jnp.float32),
    mesh=_mesh,
    scratch_types=(
        [pltpu.VMEM((2, _CH), jnp.int32)] * 4
        + [pltpu.VMEM((_CH, _H), jnp.float32)] * 4
        + [pltpu.VMEM_SHARED((_NPAD, _H), jnp.float32)]
        + [pltpu.SemaphoreType.DMA] * 12
    ),
)


_BM = 1000


def _mlp_body(x_ref, agg_ref, w1_ref, b1_ref, w2_ref, b2_ref, g_ref, be_ref, out_ref):
    h = x_ref[...] + agg_ref[0] + agg_ref[1]
    h = jnp.maximum(jnp.dot(h, w1_ref[...], preferred_element_type=jnp.float32) + b1_ref[...], 0.0)
    h = jnp.maximum(jnp.dot(h, w2_ref[...], preferred_element_type=jnp.float32) + b2_ref[...], 0.0)
    out_ref[...] = h * (g_ref[...] * _INV_BN) + be_ref[...]


_mlp = pl.pallas_call(
    _mlp_body,
    grid=(_N // _BM,),
    in_specs=[
        pl.BlockSpec((_BM, _H), lambda i: (i, 0)),
        pl.BlockSpec((_NC, _BM, _H), lambda i: (0, i, 0)),
        pl.BlockSpec((_H, _H), lambda i: (0, 0)),
        pl.BlockSpec((1, _H), lambda i: (0, 0)),
        pl.BlockSpec((_H, _H), lambda i: (0, 0)),
        pl.BlockSpec((1, _H), lambda i: (0, 0)),
        pl.BlockSpec((1, _H), lambda i: (0, 0)),
        pl.BlockSpec((1, _H), lambda i: (0, 0)),
    ],
    out_specs=pl.BlockSpec((_BM, _H), lambda i: (i, 0)),
    out_shape=jax.ShapeDtypeStruct((_N, _H), jnp.float32),
)


def _pool_body(adj_ref, batch_ref, x1_ref, x2_ref, x3_ref, wm1_ref, bm1_ref,
               wm2_ref, bm2_ref, out_ref, pooled_s, cnt_s):
    i = pl.program_id(0)

    @pl.when(i == 0)
    def _():
        pooled_s[...] = jnp.zeros_like(pooled_s)
        cnt_s[...] = jnp.zeros_like(cnt_s)

    onehot = (batch_ref[...] == lax.broadcasted_iota(jnp.int32, (_BM, _G), 1)).astype(jnp.float32)
    h = jnp.concatenate([x1_ref[...], x2_ref[...], x3_ref[...]], axis=1)
    pooled_s[...] += lax.dot_general(
        onehot, h, (((0,), (0,)), ((), ())), preferred_element_type=jnp.float32)
    cnt_s[...] += lax.dot_general(
        onehot, jnp.ones((_BM, 1), jnp.float32), (((0,), (0,)), ((), ())),
        preferred_element_type=jnp.float32)

    @pl.when(i == _N // _BM - 1)
    def _():
        cnt = cnt_s[...] + adj_ref[0, 0]
        mean = pooled_s[...] / jnp.maximum(cnt, 1.0)
        hm = jnp.maximum(
            jnp.dot(mean, wm1_ref[...], preferred_element_type=jnp.float32) + bm1_ref[...], 0.0)
        out_ref[...] = jnp.dot(hm, wm2_ref[...], preferred_element_type=jnp.float32) + bm2_ref[...]


_pool = pl.pallas_call(
    _pool_body,
    grid=(_N // _BM,),
    in_specs=[
        pl.BlockSpec((1, 1), lambda i: (0, 0)),
        pl.BlockSpec((_BM, 1), lambda i: (i, 0)),
        pl.BlockSpec((_BM, _H), lambda i: (i, 0)),
        pl.BlockSpec((_BM, _H), lambda i: (i, 0)),
        pl.BlockSpec((_BM, _H), lambda i: (i, 0)),
        pl.BlockSpec((3 * _H, _H), lambda i: (0, 0)),
        pl.BlockSpec((1, _H), lambda i: (0, 0)),
        pl.BlockSpec((_H, 1), lambda i: (0, 0)),
        pl.BlockSpec((1, 1), lambda i: (0, 0)),
    ],
    out_specs=pl.BlockSpec((_G, 1), lambda i: (0, 0)),
    out_shape=jax.ShapeDtypeStruct((_G, 1), jnp.float32),
    scratch_shapes=[
        pltpu.VMEM((_G, 3 * _H), jnp.float32),
        pltpu.VMEM((_G, 1), jnp.float32),
    ],
)


def kernel(num_nodes, z, edge_index, batch, z_table,
           W1_0, b1_0, W2_0, b2_0, g_0, be_0,
           W1_1, b1_1, W2_1, b2_1, g_1, be_1,
           W1_2, b1_2, W2_2, b2_2, g_2, be_2,
           Wm1, bm1, Wm2, bm2):
    ei = edge_index.astype(jnp.int32).reshape(2, _CHUNKS, _CH).transpose(1, 0, 2)
    zeros = jnp.zeros((_NPAD, _H), jnp.float32)

    x = _embed(z.astype(jnp.int32), z_table)

    layer_params = [
        (W1_0, b1_0, W2_0, b2_0, g_0, be_0),
        (W1_1, b1_1, W2_1, b2_1, g_1, be_1),
        (W1_2, b1_2, W2_2, b2_2, g_2, be_2),
    ]
    xs = []
    for (w1, b1, w2, b2, g, be) in layer_params:
        agg = _agg(x, ei, zeros)
        x = _mlp(x, agg, w1, b1.reshape(1, _H), w2, b2.reshape(1, _H),
                 g.reshape(1, _H), be.reshape(1, _H))
        xs.append(x)

    adj = (jnp.asarray(num_nodes, jnp.int32) - _N).astype(jnp.float32).reshape(1, 1)
    out = _pool(adj, batch.astype(jnp.int32).reshape(_N, 1), xs[0], xs[1], xs[2],
                Wm1, bm1.reshape(1, _H), Wm2, bm2.reshape(1, 1))
    return out


# speedup vs baseline: 1.3844x; 1.0054x over previous
import math

import jax
import jax.numpy as jnp
from jax import lax
from jax.experimental import pallas as pl
from jax.experimental.pallas import tpu as pltpu
from jax.experimental.pallas import tpu_sc as plsc

_N = 10000
_H = 128
_E = 320000
_G = 128
_BN_EPS = 1e-5
_INV_BN = 1.0 / math.sqrt(1.0 + _BN_EPS)

_NC, _NS = 2, 16
_NW = _NC * _NS

_CH = 50
_CHUNKS = _E // _CH
_CPT = _CHUNKS // _NW
_NPAD = 10240

_EMB_CH = 80
_EMB_CHUNKS = _N // _EMB_CH
_EMB_PER_W = -(-_EMB_CHUNKS // _NW)

_mesh = plsc.VectorSubcoreMesh(
    core_axis_name="c", subcore_axis_name="s", num_cores=_NC, num_subcores=_NS
)


def _embed_body(z_hbm, tab_hbm, out_hbm, idx_v, rows_v, sem):
    wid = lax.axis_index("s") * _NC + lax.axis_index("c")
    for k in range(_EMB_PER_W):
        cid = wid * _EMB_PER_W + k

        @pl.when(cid < _EMB_CHUNKS)
        def _():
            off = pl.multiple_of(cid * _EMB_CH, _EMB_CH)
            pltpu.sync_copy(z_hbm.at[pl.ds(off, _EMB_CH)], idx_v)
            pltpu.async_copy(tab_hbm.at[idx_v], rows_v, sem).wait()
            pltpu.sync_copy(rows_v, out_hbm.at[pl.ds(off, _EMB_CH)])


_embed = pl.kernel(
    _embed_body,
    out_type=jax.ShapeDtypeStruct((_N, _H), jnp.float32),
    mesh=_mesh,
    scratch_types=[
        pltpu.VMEM((_EMB_CH,), jnp.int32),
        pltpu.VMEM((_EMB_CH, _H), jnp.float32),
        pltpu.SemaphoreType.DMA,
    ],
)


def _agg_body(x_hbm, ei_hbm, zero_hbm, out_hbm, *refs):
    ebuf = refs[0:4]
    rows = refs[4:8]
    agg_sh = refs[8]
    esem = refs[9:13]
    gsem = refs[13:17]
    ssem = refs[17:21]

    c = lax.axis_index("c")
    s = lax.axis_index("s")

    @pl.when(s == 0)
    def _():
        pltpu.sync_copy(zero_hbm, agg_sh)

    plsc.subcore_barrier()

    chunk0 = (c * _NS + s) * _CPT

    def load_idx(g, j):
        return pltpu.async_copy(ei_hbm.at[chunk0 + g], ebuf[j], esem[j])

    def gather(g, j):
        return pltpu.async_copy(x_hbm.at[ebuf[j].at[0]], rows[j], gsem[j])

    def wait_gather(j):
        pltpu.make_async_copy(x_hbm.at[ebuf[j].at[0]], rows[j], gsem[j]).wait()

    def scatter(j):
        return pltpu.async_copy(rows[j], agg_sh.at[ebuf[j].at[1]], ssem[j], add=True)

    def wait_scatter(j):
        pltpu.make_async_copy(rows[j], agg_sh.at[ebuf[j].at[1]], ssem[j]).wait()

    def wait_idx(j):
        pltpu.make_async_copy(ei_hbm.at[chunk0], ebuf[j], esem[j]).wait()

    load_idx(0, 0)
    load_idx(1, 1)
    wait_idx(0)
    gather(0, 0)
    wait_idx(1)
    gather(1, 1)

    @pl.loop(0, _CPT, step=4)
    def _(g4):
        for j in range(4):
            g = g4 + j
            j2 = (j + 2) % 4

            if j < 2:
                @pl.when(g >= 2)
                def _():
                    wait_scatter(j2)

                @pl.when(g + 2 < _CPT)
                def _():
                    load_idx(g + 2, j2)
            else:
                wait_scatter(j2)

                @pl.when(g + 2 < _CPT)
                def _():
                    load_idx(g + 2, j2)

            wait_gather(j)
            scatter(j)

            @pl.when(g + 2 < _CPT)
            def _():
                wait_idx(j2)
                gather(g + 2, j2)

    wait_scatter((_CPT - 2) % 4)
    wait_scatter((_CPT - 1) % 4)

    plsc.subcore_barrier()
    rpt = _NPAD // _NS
    pltpu.sync_copy(
        agg_sh.at[pl.ds(s * rpt, rpt)],
        out_hbm.at[c, pl.ds(s * rpt, rpt)],
    )


_agg = pl.kernel(
    _agg_body,
    out_type=jax.ShapeDtypeStruct((_NC, _NPAD, _H), jnp.float32),
    mesh=_mesh,
    scratch_types=(
        [pltpu.VMEM((2, _CH), jnp.int32)] * 4
        + [pltpu.VMEM((_CH, _H), jnp.float32)] * 4
        + [pltpu.VMEM_SHARED((_NPAD, _H), jnp.float32)]
        + [pltpu.SemaphoreType.DMA] * 12
    ),
)


_BM = 1000


def _mlp_body(x_ref, agg_ref, batch_ref, w1_ref, b1_ref, w2_ref, b2_ref,
              g_ref, be_ref, out_ref, pp_ref, cnt_ref, pp_s, cnt_s):
    i = pl.program_id(0)
    h = x_ref[...] + agg_ref[0] + agg_ref[1]
    h = jnp.maximum(jnp.dot(h, w1_ref[...], preferred_element_type=jnp.float32) + b1_ref[...], 0.0)
    h = jnp.maximum(jnp.dot(h, w2_ref[...], preferred_element_type=jnp.float32) + b2_ref[...], 0.0)
    y = h * (g_ref[...] * _INV_BN) + be_ref[...]
    out_ref[...] = y

    onehot = (batch_ref[...] == lax.broadcasted_iota(jnp.int32, (_BM, _G), 1)).astype(jnp.float32)

    @pl.when(i == 0)
    def _():
        pp_s[...] = jnp.zeros_like(pp_s)
        cnt_s[...] = jnp.zeros_like(cnt_s)

    pp_s[...] += lax.dot_general(
        onehot, y, (((0,), (0,)), ((), ())), preferred_element_type=jnp.float32)
    cnt_s[...] += lax.dot_general(
        onehot, jnp.ones((_BM, 1), jnp.float32), (((0,), (0,)), ((), ())),
        preferred_element_type=jnp.float32)

    @pl.when(i == _N // _BM - 1)
    def _():
        pp_ref[...] = pp_s[...]
        cnt_ref[...] = cnt_s[...]


_mlp = pl.pallas_call(
    _mlp_body,
    grid=(_N // _BM,),
    in_specs=[
        pl.BlockSpec((_BM, _H), lambda i: (i, 0)),
        pl.BlockSpec((_NC, _BM, _H), lambda i: (0, i, 0)),
        pl.BlockSpec((_BM, 1), lambda i: (i, 0)),
        pl.BlockSpec((_H, _H), lambda i: (0, 0)),
        pl.BlockSpec((1, _H), lambda i: (0, 0)),
        pl.BlockSpec((_H, _H), lambda i: (0, 0)),
        pl.BlockSpec((1, _H), lambda i: (0, 0)),
        pl.BlockSpec((1, _H), lambda i: (0, 0)),
        pl.BlockSpec((1, _H), lambda i: (0, 0)),
    ],
    out_specs=[
        pl.BlockSpec((_BM, _H), lambda i: (i, 0)),
        pl.BlockSpec((_G, _H), lambda i: (0, 0)),
        pl.BlockSpec((_G, 1), lambda i: (0, 0)),
    ],
    out_shape=[
        jax.ShapeDtypeStruct((_N, _H), jnp.float32),
        jax.ShapeDtypeStruct((_G, _H), jnp.float32),
        jax.ShapeDtypeStruct((_G, 1), jnp.float32),
    ],
    scratch_shapes=[
        pltpu.VMEM((_G, _H), jnp.float32),
        pltpu.VMEM((_G, 1), jnp.float32),
    ],
)


def _head_body(adj_ref, pp0_ref, pp1_ref, pp2_ref, cnt_ref, wm1_ref, bm1_ref,
               wm2_ref, bm2_ref, out_ref):
    cnt = jnp.maximum(cnt_ref[...] + adj_ref[0, 0], 1.0)
    pooled = jnp.concatenate([pp0_ref[...], pp1_ref[...], pp2_ref[...]], axis=1) / cnt
    hm = jnp.maximum(
        jnp.dot(pooled, wm1_ref[...], preferred_element_type=jnp.float32) + bm1_ref[...], 0.0)
    out_ref[...] = jnp.dot(hm, wm2_ref[...], preferred_element_type=jnp.float32) + bm2_ref[...]


_head = pl.pallas_call(
    _head_body,
    out_shape=jax.ShapeDtypeStruct((_G, 1), jnp.float32),
)


def kernel(num_nodes, z, edge_index, batch, z_table,
           W1_0, b1_0, W2_0, b2_0, g_0, be_0,
           W1_1, b1_1, W2_1, b2_1, g_1, be_1,
           W1_2, b1_2, W2_2, b2_2, g_2, be_2,
           Wm1, bm1, Wm2, bm2):
    ei = edge_index.astype(jnp.int32).reshape(2, _CHUNKS, _CH).transpose(1, 0, 2)
    zeros = jnp.zeros((_NPAD, _H), jnp.float32)

    x = _embed(z.astype(jnp.int32), z_table)

    layer_params = [
        (W1_0, b1_0, W2_0, b2_0, g_0, be_0),
        (W1_1, b1_1, W2_1, b2_1, g_1, be_1),
        (W1_2, b1_2, W2_2, b2_2, g_2, be_2),
    ]
    batch2 = batch.astype(jnp.int32).reshape(_N, 1)
    pps = []
    cnt = None
    for (w1, b1, w2, b2, g, be) in layer_params:
        agg = _agg(x, ei, zeros)
        x, pp, cnt = _mlp(x, agg, batch2, w1, b1.reshape(1, _H), w2,
                          b2.reshape(1, _H), g.reshape(1, _H), be.reshape(1, _H))
        pps.append(pp)

    adj = (jnp.asarray(num_nodes, jnp.int32) - _N).astype(jnp.float32).reshape(1, 1)
    out = _head(adj, pps[0], pps[1], pps[2], cnt,
                Wm1, bm1.reshape(1, _H), Wm2, bm2.reshape(1, 1))
    return out


# speedup vs baseline: 1.3908x; 1.0046x over previous
import math

import jax
import jax.numpy as jnp
from jax import lax
from jax.experimental import pallas as pl
from jax.experimental.pallas import tpu as pltpu
from jax.experimental.pallas import tpu_sc as plsc

_N = 10000
_H = 128
_E = 320000
_G = 128
_BN_EPS = 1e-5
_INV_BN = 1.0 / math.sqrt(1.0 + _BN_EPS)

_NC, _NS = 2, 16
_NW = _NC * _NS

_CH = 50
_CHUNKS = _E // _CH
_CPT = _CHUNKS // _NW
_NPAD = 10240

_EMB_CH = 80
_EMB_CHUNKS = _N // _EMB_CH
_EMB_PER_W = -(-_EMB_CHUNKS // _NW)

_mesh = plsc.VectorSubcoreMesh(
    core_axis_name="c", subcore_axis_name="s", num_cores=_NC, num_subcores=_NS
)


def _embed_body(z_hbm, tab_hbm, out_hbm, idx_v, rows_v, sem):
    wid = lax.axis_index("s") * _NC + lax.axis_index("c")
    for k in range(_EMB_PER_W):
        cid = wid * _EMB_PER_W + k

        @pl.when(cid < _EMB_CHUNKS)
        def _():
            off = pl.multiple_of(cid * _EMB_CH, _EMB_CH)
            pltpu.sync_copy(z_hbm.at[pl.ds(off, _EMB_CH)], idx_v)
            pltpu.async_copy(tab_hbm.at[idx_v], rows_v, sem).wait()
            pltpu.sync_copy(rows_v, out_hbm.at[pl.ds(off, _EMB_CH)])


_embed = pl.kernel(
    _embed_body,
    out_type=jax.ShapeDtypeStruct((_N, _H), jnp.float32),
    mesh=_mesh,
    scratch_types=[
        pltpu.VMEM((_EMB_CH,), jnp.int32),
        pltpu.VMEM((_EMB_CH, _H), jnp.float32),
        pltpu.SemaphoreType.DMA,
    ],
)


def _agg_body(x_hbm, ei_hbm, zero_hbm, out_hbm, *refs):
    ebuf = refs[0:4]
    rows = refs[4:8]
    agg_sh = refs[8]
    esem = refs[9:13]
    gsem = refs[13:17]
    ssem = refs[17:21]

    c = lax.axis_index("c")
    s = lax.axis_index("s")

    chunk0 = (c * _NS + s) * _CPT
    rpt = _NPAD // _NS

    def load_idx(g, j):
        return pltpu.async_copy(ei_hbm.at[chunk0 + g], ebuf[j], esem[j])

    def gather(g, j):
        return pltpu.async_copy(x_hbm.at[ebuf[j].at[0]], rows[j], gsem[j])

    def wait_gather(j):
        pltpu.make_async_copy(x_hbm.at[ebuf[j].at[0]], rows[j], gsem[j]).wait()

    def scatter(j):
        return pltpu.async_copy(rows[j], agg_sh.at[ebuf[j].at[1]], ssem[j], add=True)

    def wait_scatter(j):
        pltpu.make_async_copy(rows[j], agg_sh.at[ebuf[j].at[1]], ssem[j]).wait()

    def wait_idx(j):
        pltpu.make_async_copy(ei_hbm.at[chunk0], ebuf[j], esem[j]).wait()

    load_idx(0, 0)
    load_idx(1, 1)
    wait_idx(0)
    gather(0, 0)
    wait_idx(1)
    gather(1, 1)

    pltpu.sync_copy(zero_hbm.at[pl.ds(s * rpt, rpt)],
                    agg_sh.at[pl.ds(s * rpt, rpt)])
    plsc.subcore_barrier()

    @pl.loop(0, _CPT, step=4)
    def _(g4):
        for j in range(4):
            g = g4 + j
            j2 = (j + 2) % 4

            if j < 2:
                @pl.when(g >= 2)
                def _():
                    wait_scatter(j2)

                @pl.when(g + 2 < _CPT)
                def _():
                    load_idx(g + 2, j2)
            else:
                wait_scatter(j2)

                @pl.when(g + 2 < _CPT)
                def _():
                    load_idx(g + 2, j2)

            wait_gather(j)
            scatter(j)

            @pl.when(g + 2 < _CPT)
            def _():
                wait_idx(j2)
                gather(g + 2, j2)

    wait_scatter((_CPT - 2) % 4)
    wait_scatter((_CPT - 1) % 4)

    plsc.subcore_barrier()
    pltpu.sync_copy(
        agg_sh.at[pl.ds(s * rpt, rpt)],
        out_hbm.at[c, pl.ds(s * rpt, rpt)],
    )


_agg = pl.kernel(
    _agg_body,
    out_type=jax.ShapeDtypeStruct((_NC, _NPAD, _H), jnp.float32),
    mesh=_mesh,
    scratch_types=(
        [pltpu.VMEM((2, _CH), jnp.int32)] * 4
        + [pltpu.VMEM((_CH, _H), jnp.float32)] * 4
        + [pltpu.VMEM_SHARED((_NPAD, _H), jnp.float32)]
        + [pltpu.SemaphoreType.DMA] * 12
    ),
)


_BM = 1000


def _mlp_body(x_ref, agg_ref, batch_ref, w1_ref, b1_ref, w2_ref, b2_ref,
              g_ref, be_ref, out_ref, pp_ref, cnt_ref, pp_s, cnt_s):
    i = pl.program_id(0)
    h = x_ref[...] + agg_ref[0] + agg_ref[1]
    h = jnp.maximum(jnp.dot(h, w1_ref[...], preferred_element_type=jnp.float32) + b1_ref[...], 0.0)
    h = jnp.maximum(jnp.dot(h, w2_ref[...], preferred_element_type=jnp.float32) + b2_ref[...], 0.0)
    y = h * (g_ref[...] * _INV_BN) + be_ref[...]
    out_ref[...] = y

    onehot = (batch_ref[...] == lax.broadcasted_iota(jnp.int32, (_BM, _G), 1)).astype(jnp.float32)

    @pl.when(i == 0)
    def _():
        pp_s[...] = jnp.zeros_like(pp_s)
        cnt_s[...] = jnp.zeros_like(cnt_s)

    pp_s[...] += lax.dot_general(
        onehot, y, (((0,), (0,)), ((), ())), preferred_element_type=jnp.float32)
    cnt_s[...] += lax.dot_general(
        onehot, jnp.ones((_BM, 1), jnp.float32), (((0,), (0,)), ((), ())),
        preferred_element_type=jnp.float32)

    @pl.when(i == _N // _BM - 1)
    def _():
        pp_ref[...] = pp_s[...]
        cnt_ref[...] = cnt_s[...]


_mlp = pl.pallas_call(
    _mlp_body,
    grid=(_N // _BM,),
    in_specs=[
        pl.BlockSpec((_BM, _H), lambda i: (i, 0)),
        pl.BlockSpec((_NC, _BM, _H), lambda i: (0, i, 0)),
        pl.BlockSpec((_BM, 1), lambda i: (i, 0)),
        pl.BlockSpec((_H, _H), lambda i: (0, 0)),
        pl.BlockSpec((1, _H), lambda i: (0, 0)),
        pl.BlockSpec((_H, _H), lambda i: (0, 0)),
        pl.BlockSpec((1, _H), lambda i: (0, 0)),
        pl.BlockSpec((1, _H), lambda i: (0, 0)),
        pl.BlockSpec((1, _H), lambda i: (0, 0)),
    ],
    out_specs=[
        pl.BlockSpec((_BM, _H), lambda i: (i, 0)),
        pl.BlockSpec((_G, _H), lambda i: (0, 0)),
        pl.BlockSpec((_G, 1), lambda i: (0, 0)),
    ],
    out_shape=[
        jax.ShapeDtypeStruct((_N, _H), jnp.float32),
        jax.ShapeDtypeStruct((_G, _H), jnp.float32),
        jax.ShapeDtypeStruct((_G, 1), jnp.float32),
    ],
    scratch_shapes=[
        pltpu.VMEM((_G, _H), jnp.float32),
        pltpu.VMEM((_G, 1), jnp.float32),
    ],
)


def _head_body(adj_ref, pp0_ref, pp1_ref, pp2_ref, cnt_ref, wm1_ref, bm1_ref,
               wm2_ref, bm2_ref, out_ref):
    cnt = jnp.maximum(cnt_ref[...] + adj_ref[0, 0], 1.0)
    pooled = jnp.concatenate([pp0_ref[...], pp1_ref[...], pp2_ref[...]], axis=1) / cnt
    hm = jnp.maximum(
        jnp.dot(pooled, wm1_ref[...], preferred_element_type=jnp.float32) + bm1_ref[...], 0.0)
    out_ref[...] = jnp.dot(hm, wm2_ref[...], preferred_element_type=jnp.float32) + bm2_ref[...]


_head = pl.pallas_call(
    _head_body,
    out_shape=jax.ShapeDtypeStruct((_G, 1), jnp.float32),
)


def kernel(num_nodes, z, edge_index, batch, z_table,
           W1_0, b1_0, W2_0, b2_0, g_0, be_0,
           W1_1, b1_1, W2_1, b2_1, g_1, be_1,
           W1_2, b1_2, W2_2, b2_2, g_2, be_2,
           Wm1, bm1, Wm2, bm2):
    ei = edge_index.astype(jnp.int32).reshape(2, _CHUNKS, _CH).transpose(1, 0, 2)
    zeros = jnp.zeros((_NPAD, _H), jnp.float32)

    x = _embed(z.astype(jnp.int32), z_table)

    layer_params = [
        (W1_0, b1_0, W2_0, b2_0, g_0, be_0),
        (W1_1, b1_1, W2_1, b2_1, g_1, be_1),
        (W1_2, b1_2, W2_2, b2_2, g_2, be_2),
    ]
    batch2 = batch.astype(jnp.int32).reshape(_N, 1)
    pps = []
    cnt = None
    for (w1, b1, w2, b2, g, be) in layer_params:
        agg = _agg(x, ei, zeros)
        x, pp, cnt = _mlp(x, agg, batch2, w1, b1.reshape(1, _H), w2,
                          b2.reshape(1, _H), g.reshape(1, _H), be.reshape(1, _H))
        pps.append(pp)

    adj = (jnp.asarray(num_nodes, jnp.int32) - _N).astype(jnp.float32).reshape(1, 1)
    out = _head(adj, pps[0], pps[1], pps[2], cnt,
                Wm1, bm1.reshape(1, _H), Wm2, bm2.reshape(1, 1))
    return out
